# baseline (speedup 1.0000x reference)
import functools

import jax
import jax.numpy as jnp
from jax import lax
from jax.experimental import pallas as pl
from jax.experimental.pallas import tpu as pltpu
from jax.experimental.pallas import tpu_sc as plsc

L = 200
D = 64
B = 4096
V = 1000000
LANES = 16
NWIN = V // 128


def _make_call1():
    info = plsc.get_sparse_core_info()
    nc = info.num_cores
    nw = nc * info.num_subcores
    nj = (NWIN + nw - 1) // nw

    mesh = plsc.VectorSubcoreMesh(core_axis_name="c", subcore_axis_name="s")

    @functools.partial(
        pl.kernel,
        mesh=mesh,
        out_type=jax.ShapeDtypeStruct((V // 2, 128), jnp.float32),
        scratch_types=[
            pltpu.VMEM((2, D, 128), jnp.float32),
            pltpu.VMEM((2, D, 128), jnp.float32),
            pltpu.SemaphoreType.DMA,
            pltpu.SemaphoreType.DMA,
        ],
        compiler_params=pltpu.CompilerParams(
            use_tc_tiling_on_sc=True, needs_layout_passes=False),
    )
    def k(tokT_hbm, out_hbm, inw, outw, isem, osem):
        wid = lax.axis_index("s") * nc + lax.axis_index("c")

        rows_c = [lax.iota(jnp.int32, LANES) + fc * LANES for fc in range(4)]

        def cw_of(j):
            return wid + j * nw

        def issue_in(j, s):
            pltpu.async_copy(
                tokT_hbm.at[:, pl.ds(cw_of(j) * 128, 128)], inw.at[s], isem)

        def wait_in(j, s):
            pltpu.make_async_copy(
                tokT_hbm.at[:, pl.ds(cw_of(j) * 128, 128)], inw.at[s],
                isem).wait()

        def issue_out(j, s):
            pltpu.async_copy(
                outw.at[s], out_hbm.at[pl.ds(cw_of(j) * 64, 64)], osem)

        def wait_out(j, s):
            pltpu.make_async_copy(
                outw.at[s], out_hbm.at[pl.ds(cw_of(j) * 64, 64)], osem).wait()

        def transpose_window(s):
            def u_body(u, c):
                for half in range(2):
                    t = 2 * u + half
                    cols = jnp.broadcast_to(t.astype(jnp.int32), (LANES,))
                    for fc in range(4):
                        g = plsc.load_gather(inw.at[s], [rows_c[fc], cols])
                        outw[s, u, pl.ds(half * 64 + fc * LANES, LANES)] = g
                return c

            lax.fori_loop(0, D, u_body, 0)

        @pl.when(cw_of(0) < NWIN)
        def _():
            issue_in(0, 0)

        def j_body(j, c):
            s = lax.rem(j, 2)

            @pl.when((j >= 2) & (cw_of(j - 2) < NWIN))
            def _():
                wait_out(j - 2, s)

            @pl.when(cw_of(j) < NWIN)
            def _():
                @pl.when(cw_of(j + 1) < NWIN)
                def _():
                    issue_in(j + 1, 1 - s)
                wait_in(j, s)
                transpose_window(s)
                issue_out(j, s)

            return c

        lax.fori_loop(0, nj, j_body, 0)

        def drain(j, c):
            @pl.when(cw_of(j) < NWIN)
            def _():
                wait_out(j, lax.rem(j, 2))
            return c

        lax.fori_loop(nj - 2, nj, drain, 0)

    return k


def _make_call2():
    info = plsc.get_sparse_core_info()
    nc = info.num_cores
    nw = nc * info.num_subcores
    bpw = B // nw

    mesh = plsc.VectorSubcoreMesh(core_axis_name="c", subcore_axis_name="s")

    @functools.partial(
        pl.kernel,
        mesh=mesh,
        out_type=jax.ShapeDtypeStruct((L, 8, nw, 8, 128), jnp.float32),
        scratch_types=[
            pltpu.VMEM((bpw, L), jnp.int32),
            pltpu.VMEM((L, bpw), jnp.int32),
            pltpu.VMEM((L, D), jnp.float32),
            pltpu.VMEM((2, bpw, D), jnp.float32),
            pltpu.VMEM((2, 8, 8, 128), jnp.float32),
            pltpu.SemaphoreType.DMA,
            pltpu.SemaphoreType.DMA,
        ],
        compiler_params=pltpu.CompilerParams(
            use_tc_tiling_on_sc=False, needs_layout_passes=False),
    )
    def k(idx_hbm, tok_hbm, pos_hbm, out_hbm, islab, idxT, pos_v, rows_v,
          obuf, gsem, osem):
        w = lax.axis_index("s") * nc + lax.axis_index("c")
        pltpu.sync_copy(pos_hbm, pos_v)
        pltpu.sync_copy(idx_hbm.at[pl.ds(w * bpw, bpw)], islab)

        rows_c = [lax.iota(jnp.int32, LANES) + bc * LANES for bc in range(8)]

        def tpose_idx(l, c):
            cols = jnp.broadcast_to(l.astype(jnp.int32), (LANES,))
            for bc in range(8):
                idxT[l, pl.ds(bc * LANES, LANES)] = plsc.load_gather(
                    islab, [rows_c[bc], cols])
            return c

        lax.fori_loop(0, L, tpose_idx, 0)

        def issue_gather(l, s):
            pltpu.async_copy(tok_hbm.at[idxT.at[l]], rows_v.at[s], gsem)

        def wait_gather(l, s):
            pltpu.make_async_copy(
                tok_hbm.at[idxT.at[l]], rows_v.at[s], gsem).wait()

        def issue_store(l, s):
            for fb in range(8):
                pltpu.async_copy(obuf.at[s, fb], out_hbm.at[l, fb, w], osem)

        def wait_store(l, s):
            for fb in range(8):
                pltpu.make_async_copy(
                    obuf.at[s, fb], out_hbm.at[l, fb, w], osem).wait()

        def transpose_add(l, s):
            lrow = jnp.broadcast_to(l.astype(jnp.int32), (LANES,))

            def f_body(fi, c):
                for fb in range(8):
                    f = fb * 8 + fi
                    cols = jnp.broadcast_to(f.astype(jnp.int32), (LANES,))
                    p = plsc.load_gather(pos_v, [lrow, cols])
                    for bc in range(8):
                        g = plsc.load_gather(rows_v.at[s], [rows_c[bc], cols])
                        obuf[s, fb, fi, pl.ds(bc * LANES, LANES)] = g + p
                return c

            lax.fori_loop(0, 8, f_body, 0)

        issue_gather(0, 0)

        def l_body(l, c):
            s = lax.rem(l, 2)

            @pl.when(l + 1 < L)
            def _():
                issue_gather(l + 1, 1 - s)
            wait_gather(l, s)
            @pl.when(l >= 2)
            def _():
                wait_store(l - 2, s)
            transpose_add(l, s)
            issue_store(l, s)
            return c

        lax.fori_loop(0, L, l_body, 0)
        wait_store(L - 2, 0)
        wait_store(L - 1, 1)

    return k


_call1 = _make_call1()
_call2 = _make_call2()


@jax.jit
def kernel(inputs, token_table, position_table):
    tokT = jnp.swapaxes(token_table, 0, 1)
    d = _call1(tokT)
    tail = token_table[NWIN * 128:, :].reshape(32, 128)
    d = lax.dynamic_update_slice(d, tail, (NWIN * 64, 0))
    tok_dense = d.reshape(V, D)
    a5 = _call2(inputs.astype(jnp.int32), tok_dense, position_table)
    out = a5.transpose(2, 4, 0, 1, 3)
    return out.reshape(B, L, D)


# speedup vs baseline: 1.4127x; 1.4127x over previous
import functools

import jax
import jax.numpy as jnp
from jax import lax
from jax.experimental import pallas as pl
from jax.experimental.pallas import tpu as pltpu
from jax.experimental.pallas import tpu_sc as plsc

L = 200
D = 64
B = 4096
V = 1000000
LANES = 16
NWIN = V // 128


def _make_call1():
    info = plsc.get_sparse_core_info()
    nc = info.num_cores
    nw = nc * info.num_subcores
    nj = (NWIN + nw - 1) // nw

    mesh = plsc.VectorSubcoreMesh(core_axis_name="c", subcore_axis_name="s")

    @functools.partial(
        pl.kernel,
        mesh=mesh,
        out_type=jax.ShapeDtypeStruct((V // 2, 128), jnp.float32),
        scratch_types=[
            pltpu.VMEM((2, D, 129), jnp.float32),
            pltpu.VMEM((2, D, 128), jnp.float32),
            pltpu.SemaphoreType.DMA,
            pltpu.SemaphoreType.DMA,
        ],
        compiler_params=pltpu.CompilerParams(
            use_tc_tiling_on_sc=True, needs_layout_passes=False),
    )
    def k(tokT_hbm, out_hbm, inw, outw, isem, osem):
        wid = lax.axis_index("s") * nc + lax.axis_index("c")

        rows_c = [lax.iota(jnp.int32, LANES) + fc * LANES for fc in range(4)]

        def cw_of(j):
            return wid + j * nw

        def issue_in(j, s):
            pltpu.async_copy(
                tokT_hbm.at[:, pl.ds(cw_of(j) * 128, 128)],
                inw.at[s, :, pl.ds(0, 128)], isem)

        def wait_in(j, s):
            pltpu.make_async_copy(
                tokT_hbm.at[:, pl.ds(cw_of(j) * 128, 128)],
                inw.at[s, :, pl.ds(0, 128)], isem).wait()

        def issue_out(j, s):
            pltpu.async_copy(
                outw.at[s], out_hbm.at[pl.ds(cw_of(j) * 64, 64)], osem)

        def wait_out(j, s):
            pltpu.make_async_copy(
                outw.at[s], out_hbm.at[pl.ds(cw_of(j) * 64, 64)], osem).wait()

        def transpose_window(s):
            def u_body(u, c):
                for half in range(2):
                    t = 2 * u + half
                    cols = jnp.broadcast_to(t.astype(jnp.int32), (LANES,))
                    for fc in range(4):
                        g = plsc.load_gather(inw.at[s], [rows_c[fc], cols])
                        outw[s, u, pl.ds(half * 64 + fc * LANES, LANES)] = g
                return c

            lax.fori_loop(0, D, u_body, 0)

        @pl.when(cw_of(0) < NWIN)
        def _():
            issue_in(0, 0)

        def j_body(j, c):
            s = lax.rem(j, 2)

            @pl.when((j >= 2) & (cw_of(j - 2) < NWIN))
            def _():
                wait_out(j - 2, s)

            @pl.when(cw_of(j) < NWIN)
            def _():
                @pl.when(cw_of(j + 1) < NWIN)
                def _():
                    issue_in(j + 1, 1 - s)
                wait_in(j, s)
                transpose_window(s)
                issue_out(j, s)

            return c

        lax.fori_loop(0, nj, j_body, 0)

        def drain(j, c):
            @pl.when(cw_of(j) < NWIN)
            def _():
                wait_out(j, lax.rem(j, 2))
            return c

        lax.fori_loop(nj - 2, nj, drain, 0)

    return k


def _make_call2():
    info = plsc.get_sparse_core_info()
    nc = info.num_cores
    nw = nc * info.num_subcores
    bpw = B // nw

    mesh = plsc.VectorSubcoreMesh(core_axis_name="c", subcore_axis_name="s")

    @functools.partial(
        pl.kernel,
        mesh=mesh,
        out_type=jax.ShapeDtypeStruct((L, 8, nw, 8, 128), jnp.float32),
        scratch_types=[
            pltpu.VMEM((bpw, 201), jnp.int32),
            pltpu.VMEM((L, bpw), jnp.int32),
            pltpu.VMEM((L, D), jnp.float32),
            pltpu.VMEM((2, bpw, D), jnp.float32),
            pltpu.VMEM((2, D, 129), jnp.float32),
            pltpu.SemaphoreType.DMA,
            pltpu.SemaphoreType.DMA,
        ],
        compiler_params=pltpu.CompilerParams(
            use_tc_tiling_on_sc=False, needs_layout_passes=False),
    )
    def k(idx_hbm, tok_hbm, pos_hbm, out_hbm, islab, idxT, pos_v, rows_v,
          obuf, gsem, osem):
        w = lax.axis_index("s") * nc + lax.axis_index("c")
        pltpu.sync_copy(pos_hbm, pos_v)
        pltpu.sync_copy(idx_hbm.at[pl.ds(w * bpw, bpw)],
                        islab.at[:, pl.ds(0, L)])

        rows_c = [lax.iota(jnp.int32, LANES) + bc * LANES for bc in range(8)]

        def tpose_idx(l, c):
            cols = jnp.broadcast_to(l.astype(jnp.int32), (LANES,))
            for bc in range(8):
                idxT[l, pl.ds(bc * LANES, LANES)] = plsc.load_gather(
                    islab, [rows_c[bc], cols])
            return c

        lax.fori_loop(0, L, tpose_idx, 0)

        def issue_gather(l, s):
            pltpu.async_copy(tok_hbm.at[idxT.at[l]], rows_v.at[s], gsem)

        def wait_gather(l, s):
            pltpu.make_async_copy(
                tok_hbm.at[idxT.at[l]], rows_v.at[s], gsem).wait()

        def issue_store(l, s):
            for fb in range(8):
                pltpu.async_copy(
                    obuf.at[s, pl.ds(fb * 8, 8), pl.ds(0, 128)],
                    out_hbm.at[l, fb, w], osem)

        def wait_store(l, s):
            for fb in range(8):
                pltpu.make_async_copy(
                    obuf.at[s, pl.ds(fb * 8, 8), pl.ds(0, 128)],
                    out_hbm.at[l, fb, w], osem).wait()

        frows = [lax.iota(jnp.int32, LANES) + fc * LANES for fc in range(4)]

        def transpose_add(l, s):
            pv = [pos_v[l, pl.ds(fc * LANES, LANES)] for fc in range(4)]

            def b_body(bi, c):
                cols = jnp.broadcast_to(bi.astype(jnp.int32), (LANES,))
                for fc in range(4):
                    v = rows_v[s, bi, pl.ds(fc * LANES, LANES)] + pv[fc]
                    plsc.store_scatter(obuf.at[s], [frows[fc], cols], v)
                return c

            lax.fori_loop(0, bpw, b_body, 0)

        issue_gather(0, 0)

        def l_body(l, c):
            s = lax.rem(l, 2)

            @pl.when(l + 1 < L)
            def _():
                issue_gather(l + 1, 1 - s)
            wait_gather(l, s)
            @pl.when(l >= 2)
            def _():
                wait_store(l - 2, s)
            transpose_add(l, s)
            issue_store(l, s)
            return c

        lax.fori_loop(0, L, l_body, 0)
        wait_store(L - 2, 0)
        wait_store(L - 1, 1)

    return k


_call1 = _make_call1()
_call2 = _make_call2()


@jax.jit
def kernel(inputs, token_table, position_table):
    tokT = jnp.swapaxes(token_table, 0, 1)
    d = _call1(tokT)
    tail = token_table[NWIN * 128:, :].reshape(32, 128)
    d = lax.dynamic_update_slice(d, tail, (NWIN * 64, 0))
    tok_dense = d.reshape(V, D)
    a5 = _call2(inputs.astype(jnp.int32), tok_dense, position_table)
    out = a5.transpose(2, 4, 0, 1, 3)
    return out.reshape(B, L, D)
